# baseline (speedup 1.0000x reference)
import numpy as np
import jax
import jax.numpy as jnp
from jax import lax
from jax.experimental import pallas as pl
from jax.experimental.pallas import tpu as pltpu
from jax.experimental.pallas import tpu_sc as plsc

_RADIUS = np.float32(0.1)
_GRID = 128
_OFFSET = 64
_B, _N, _D = 16, 100000, 3
_NBINS = _GRID ** 3
_HALF = _NBINS // 2
_CHUNK = 6272
_NG = _CHUNK // 16
_NI = _CHUNK // 128
_VALID15 = _N - 15 * _CHUNK
_HIST = _HALF + 128
_ZS = _HIST // 16
_NPTS_PAD = (_B - 1) * _N + 16 * _CHUNK
_BPC = _B // 2


def _neighbor_bin_map():
    m = [[x // 9 - 1, (x // 3) % 3 - 1, x % 3 - 1] for x in range(27)]
    return jnp.asarray(np.array(m), dtype=jnp.int32)


def _body(pts_hbm, keys_hbm, ids_hbm, cnt_hbm,
          pts_v, ids_v, lids_v, c0_v, c1_v, ones_v, zeros_v, hist_sh):
    cid = lax.axis_index("c")
    sid = lax.axis_index("s")
    iota = lax.iota(jnp.int32, 16)

    def init_z(i, c):
        zeros_v[pl.ds(i * 16, 16)] = jnp.zeros((16,), jnp.int32)
        return c
    lax.fori_loop(0, zeros_v.shape[0] // 16, init_z, 0)

    for u in range(8):
        ones_v[pl.ds(u * 16, 16)] = jnp.ones((16,), jnp.int32)

    def per_batch(bi, carry):
        b = cid * _BPC + bi
        pbase = pl.multiple_of(b * _N + sid * _CHUNK, 8)

        pltpu.sync_copy(pts_hbm.at[pl.ds(pbase * 3, _CHUNK * 3)], pts_v)

        tk = [jnp.clip(iota * 3 + d - 16 * j, 0, 15)
              for d in range(3) for j in range(3)]
        sel = [(iota < 6, iota < 11), (iota < 5, iota < 11), (iota < 5, iota < 10)]

        def quant(g, c):
            off = g * 48
            ks = []
            for j in range(3):
                q = pts_v[pl.ds(off + j * 16, 16)] / _RADIUS
                t = q.astype(jnp.int32)
                tf = t.astype(jnp.float32)
                adj = tf > q
                k = jnp.where(adj, t - 1, t)
                pts_v[pl.ds(off + j * 16, 16)] = jnp.where(adj, tf - 1.0, tf)
                ks.append(jnp.clip(k + _OFFSET, 0, _GRID - 1))
            lin = jnp.zeros((16,), jnp.int32)
            for d, w in enumerate((_GRID * _GRID, _GRID, 1)):
                g0 = jnp.take_along_axis(ks[0], tk[3 * d], axis=0)
                g1 = jnp.take_along_axis(ks[1], tk[3 * d + 1], axis=0)
                g2 = jnp.take_along_axis(ks[2], tk[3 * d + 2], axis=0)
                cd = jnp.where(sel[d][0], g0, jnp.where(sel[d][1], g1, g2))
                lin = lin + cd * w
            ids_v[pl.ds(g * 16, 16)] = lin
            return c
        lax.fori_loop(0, _NG, quant, 0)

        pltpu.sync_copy(pts_v, keys_hbm.at[pl.ds(pbase * 3, _CHUNK * 3)])
        pltpu.sync_copy(ids_v, ids_hbm.at[pl.ds(pbase, _CHUNK)])

        limit = _N - sid * _CHUNK
        zb = pl.multiple_of(sid * _ZS, 8)
        for h in range(2):
            ch_v = c0_v if h == 0 else c1_v
            plsc.subcore_barrier()

            def zloop(j, c):
                pltpu.sync_copy(
                    zeros_v, hist_sh.at[pl.ds(zb + j * zeros_v.shape[0],
                                              zeros_v.shape[0])])
                return c
            lax.fori_loop(0, 8, zloop, 0)
            pltpu.sync_copy(zeros_v.at[pl.ds(0, 8)],
                            hist_sh.at[pl.ds(zb + 65536, 8)])

            lo = h * _HALF

            def lloop(r, c):
                for u in range(8):
                    g16 = r * 128 + u * 16
                    v = ids_v[pl.ds(g16, 16)]
                    lid = v - lo
                    pos = g16 + iota
                    ok = (lid >= 0) & (lid < _HALF) & (pos < limit)
                    dummy = _HALF + (pos & 127)
                    lids_v[r, pl.ds(u * 16, 16)] = jnp.where(ok, lid, dummy)
                return c
            lax.fori_loop(0, _NI, lloop, 0)

            plsc.subcore_barrier()

            def sloop(r, c):
                pltpu.sync_copy(ones_v, hist_sh.at[lids_v.at[r]], add=True)
                return c
            lax.fori_loop(0, _NI, sloop, 0)
            plsc.subcore_barrier()

            def gloop(r, c):
                pltpu.sync_copy(hist_sh.at[lids_v.at[r]], ch_v.at[r])
                return c
            lax.fori_loop(0, _NI, gloop, 0)

        def cloop(r, c):
            for u in range(8):
                g16 = r * 128 + u * 16
                v = ids_v[pl.ds(g16, 16)]
                cnt = jnp.where(v < _HALF, c0_v[r, pl.ds(u * 16, 16)],
                                c1_v[r, pl.ds(u * 16, 16)])
                ids_v[pl.ds(g16, 16)] = cnt
            return c
        lax.fori_loop(0, _NI, cloop, 0)

        @pl.when(sid == 15)
        def _():
            pltpu.sync_copy(ids_v.at[pl.ds(0, _VALID15)],
                            cnt_hbm.at[pl.ds(pbase, _VALID15)])

        @pl.when(sid != 15)
        def _():
            pltpu.sync_copy(ids_v, cnt_hbm.at[pl.ds(pbase, _CHUNK)])

        return carry

    lax.fori_loop(0, _BPC, per_batch, 0)


def _make_kernel():
    mesh = plsc.VectorSubcoreMesh(core_axis_name="c", subcore_axis_name="s")
    return pl.kernel(
        _body,
        out_type=(
            jax.ShapeDtypeStruct((_NPTS_PAD * 3,), jnp.float32),
            jax.ShapeDtypeStruct((_NPTS_PAD,), jnp.int32),
            jax.ShapeDtypeStruct((_NPTS_PAD,), jnp.int32),
        ),
        mesh=mesh,
        scratch_types=[
            pltpu.VMEM((_CHUNK * 3,), jnp.float32),
            pltpu.VMEM((_CHUNK,), jnp.int32),
            pltpu.VMEM((_NI, 128), jnp.int32),
            pltpu.VMEM((_NI, 128), jnp.int32),
            pltpu.VMEM((_NI, 128), jnp.int32),
            pltpu.VMEM((128,), jnp.int32),
            pltpu.VMEM((8192,), jnp.int32),
            pltpu.VMEM_SHARED((_HIST,), jnp.int32),
        ],
    )


@jax.jit
def kernel(points):
    pts_flat = points.reshape(-1)
    pad = _NPTS_PAD * 3 - pts_flat.shape[0]
    pts_flat = jnp.concatenate([pts_flat, jnp.zeros((pad,), jnp.float32)])
    keys_f, ids_f, cnt_f = _make_kernel()(pts_flat)
    keys = keys_f[: _B * _N * 3].astype(jnp.int32).reshape(_B, _N, _D)
    ids = ids_f[: _B * _N].reshape(_B, _N)
    cnt = cnt_f[: _B * _N].reshape(_B, _N)
    return keys, ids, cnt, _neighbor_bin_map()
